# baseline (speedup 1.0000x reference)
import functools

import jax
import jax.numpy as jnp
from jax import lax
from jax.experimental import pallas as pl
from jax.experimental.pallas import tpu as pltpu
from jax.experimental.pallas import tpu_sc as plsc

N = 10000
E = 320000
D = 128
OUT = 7

NC = 2
NS = 16
NW = NC * NS
K = 128
NCH0 = 112
NCH1 = 46
E0 = NS * NCH0 * K
E1 = NS * NCH1 * K
E_PAD = E0 + E1
RB = 80
NBLK = N // RB

_MESH = plsc.VectorSubcoreMesh(core_axis_name="c", subcore_axis_name="s")


def _scale_rows(rows_v, w_v):
    def scale(g, carry):
        wv = w_v[pl.ds(g * 16, 16)]
        base = g * 16
        for j in range(16):
            wk = wv[j]
            for d in range(D // 16):
                sl = pl.ds(d * 16, 16)
                rows_v[base + j, sl] = rows_v[base + j, sl] * wk
        return carry
    lax.fori_loop(0, K // 16, scale, 0)


def _run_core(pack_hbm, w_hbm, s_hbm, acc_sh, nch, sid,
              ip0, ip1, w0, w1, rows0, rows1, si0, si1, sg0, sg1):
    def _fire_idx(c, ip, wv, si):
        pltpu.async_copy(pack_hbm.at[sid, c], ip, si)
        pltpu.async_copy(w_hbm.at[sid, c], wv, si)

    def _wait_idx(ip, wv, si):
        pltpu.make_async_copy(pack_hbm.at[sid, 0], ip, si).wait()
        pltpu.make_async_copy(w_hbm.at[sid, 0], wv, si).wait()

    _fire_idx(0, ip0, w0, si0)
    _fire_idx(1, ip1, w1, si1)
    _wait_idx(ip0, w0, si0)
    pltpu.async_copy(s_hbm.at[ip0.at[1]], rows0, sg0)

    def _slot(c, ipA, wA, rowsA, sgA, siA, ipB, wB, rowsB, sgB, siB):
        @pl.when(c + 1 < nch)
        def _fire_next_gather():
            _wait_idx(ipB, wB, siB)
            pltpu.async_copy(s_hbm.at[ipB.at[1]], rowsB, sgB)

        pltpu.make_async_copy(s_hbm.at[ipA.at[1]], rowsA, sgA).wait()
        _scale_rows(rowsA, wA)
        pltpu.sync_copy(rowsA, acc_sh.at[ipA.at[0]], add=True)

        @pl.when(c + 2 < nch)
        def _fire_next_idx():
            _fire_idx(c + 2, ipA, wA, siA)

    def pipe(i, carry):
        c = 2 * i
        _slot(c, ip0, w0, rows0, sg0, si0, ip1, w1, rows1, sg1, si1)
        _slot(c + 1, ip1, w1, rows1, sg1, si1, ip0, w0, rows0, sg0, si0)
        return carry
    lax.fori_loop(0, nch // 2, pipe, 0)


def _sc_agg_body(s_hbm, pack0_hbm, w0_hbm, pack1_hbm, w1_hbm,
                 p_hbm, acc_sh, ip0, ip1, w0, w1, rows0, rows1,
                 si0, si1, sg0, sg1):
    cid = lax.axis_index("c")
    sid = lax.axis_index("s")

    z = jnp.zeros((16,), jnp.float32)

    def _zrow(i, carry):
        for d in range(D // 16):
            rows0[i, pl.ds(d * 16, 16)] = z
        return carry
    lax.fori_loop(0, RB, _zrow, 0)
    for i in range((NBLK + NS - 1) // NS):
        blk = sid + i * NS

        @pl.when(blk < NBLK)
        def _zero():
            pltpu.sync_copy(rows0.at[pl.ds(0, RB)], acc_sh.at[pl.ds(blk * RB, RB)])
    plsc.subcore_barrier()

    @pl.when(cid == 0)
    def _core0():
        _run_core(pack0_hbm, w0_hbm, s_hbm, acc_sh, NCH0, sid,
                  ip0, ip1, w0, w1, rows0, rows1, si0, si1, sg0, sg1)

    @pl.when(cid == 1)
    def _core1():
        _run_core(pack1_hbm, w1_hbm, s_hbm, acc_sh, NCH1, sid,
                  ip0, ip1, w0, w1, rows0, rows1, si0, si1, sg0, sg1)
    plsc.subcore_barrier()

    for i in range((NBLK + NS - 1) // NS):
        blk = sid + i * NS

        @pl.when(blk < NBLK)
        def _writeback():
            r0 = blk * RB
            pltpu.sync_copy(acc_sh.at[pl.ds(r0, RB)], p_hbm.at[cid, pl.ds(r0, RB)])


_sc_agg = functools.partial(
    pl.kernel,
    out_type=jax.ShapeDtypeStruct((NC, N, D), jnp.float32),
    mesh=_MESH,
    scratch_types=[
        pltpu.VMEM_SHARED((N, D), jnp.float32),
        pltpu.VMEM((2, K), jnp.int32),
        pltpu.VMEM((2, K), jnp.int32),
        pltpu.VMEM((K,), jnp.float32),
        pltpu.VMEM((K,), jnp.float32),
        pltpu.VMEM((K, D), jnp.float32),
        pltpu.VMEM((K, D), jnp.float32),
        pltpu.SemaphoreType.DMA,
        pltpu.SemaphoreType.DMA,
        pltpu.SemaphoreType.DMA,
        pltpu.SemaphoreType.DMA,
    ],
)(_sc_agg_body)


def _tc_hl_body(x_ref, w1_ref, b1_ref, hl_ref):
    hl_ref[...] = jnp.maximum(
        jnp.dot(x_ref[...], w1_ref[...], preferred_element_type=jnp.float32)
        + b1_ref[...], 0.0)


def _tc_mid_body(p_ref, wg1_ref, wg3_ref, h1_ref, hg2_ref):
    axc = p_ref[0] + p_ref[1]
    h1_ref[...] = jnp.maximum(
        jnp.dot(axc, wg1_ref[...], preferred_element_type=jnp.float32), 0.0)
    hg2_ref[...] = jnp.maximum(
        jnp.dot(axc, wg3_ref[...], preferred_element_type=jnp.float32), 0.0)


def _tc_pre_body(hg2_ref, hl_ref, w3_ref, b3_ref, zp_ref):
    zp_ref[...] = (
        jnp.dot(hg2_ref[...], w3_ref[0:D, :], preferred_element_type=jnp.float32)
        + jnp.dot(hl_ref[...], w3_ref[D:2 * D, :],
                  preferred_element_type=jnp.float32)
        + b3_ref[...])


def _tc_final_body(p_ref, zp_ref, wg2_ref, w3a_ref, wdp_ref, bdp_ref, out_ref):
    ahc = p_ref[0] + p_ref[1]
    hg1 = jnp.maximum(
        jnp.dot(ahc, wg2_ref[...], preferred_element_type=jnp.float32), 0.0)
    z = (jnp.dot(hg1, w3a_ref[...], preferred_element_type=jnp.float32)
         + zp_ref[...])
    c = jnp.dot(z, wdp_ref[...], preferred_element_type=jnp.float32) + bdp_ref[...]
    m = jnp.max(c, axis=1, keepdims=True)
    lse = jnp.log(jnp.sum(jnp.exp(c - m), axis=1, keepdims=True))
    out_ref[...] = c - m - lse


_TC_ROWS = 1000


def kernel(x, edge_index, edge_weight, Wg1, Wg2, Wg3, W1, b1, W3, b3, Wd, bd):
    row = edge_index[0].astype(jnp.int32)
    col = edge_index[1].astype(jnp.int32)
    w = edge_weight.astype(jnp.float32)

    pad = E_PAD - E
    zpad = jnp.zeros((pad,), jnp.int32)
    row_p = jnp.concatenate([row, zpad])
    col_p = jnp.concatenate([col, zpad])
    w_p = jnp.concatenate([w, jnp.zeros((pad,), jnp.float32)])
    pack0 = jnp.stack([row_p[:E0].reshape(NS, NCH0, K),
                       col_p[:E0].reshape(NS, NCH0, K)], axis=2)
    pack1 = jnp.stack([row_p[E0:].reshape(NS, NCH1, K),
                       col_p[E0:].reshape(NS, NCH1, K)], axis=2)
    w0p = w_p[:E0].reshape(NS, NCH0, K)
    w1p = w_p[E0:].reshape(NS, NCH1, K)

    ax_p = _sc_agg(x, pack0, w0p, pack1, w1p)

    grid_rows = N // _TC_ROWS
    full = lambda i: (0, 0)
    rows_spec = pl.BlockSpec((_TC_ROWS, D), lambda i: (i, 0))
    part_spec = pl.BlockSpec((NC, _TC_ROWS, D), lambda i: (0, i, 0))
    wspec = pl.BlockSpec((D, D), full)
    bspec = pl.BlockSpec((1, D), full)

    hl = pl.pallas_call(
        _tc_hl_body,
        grid=(grid_rows,),
        in_specs=[rows_spec, wspec, bspec],
        out_specs=rows_spec,
        out_shape=jax.ShapeDtypeStruct((N, D), jnp.float32),
    )(x, W1, b1.reshape(1, D))

    h1, hg2 = pl.pallas_call(
        _tc_mid_body,
        grid=(grid_rows,),
        in_specs=[part_spec, wspec, wspec],
        out_specs=[rows_spec, rows_spec],
        out_shape=[jax.ShapeDtypeStruct((N, D), jnp.float32)] * 2,
    )(ax_p, Wg1, Wg3)

    ah_p = _sc_agg(h1, pack0, w0p, pack1, w1p)

    zp = pl.pallas_call(
        _tc_pre_body,
        grid=(grid_rows,),
        in_specs=[rows_spec, rows_spec, pl.BlockSpec((2 * D, D), full), bspec],
        out_specs=rows_spec,
        out_shape=jax.ShapeDtypeStruct((N, D), jnp.float32),
    )(hg2, hl, W3[D:3 * D], b3.reshape(1, D))

    wd_pad = jnp.zeros((D, D), jnp.float32).at[:, :OUT].set(Wd)
    bd_pad = jnp.full((1, D), -1e30, jnp.float32).at[0, :OUT].set(bd)

    out = pl.pallas_call(
        _tc_final_body,
        grid=(grid_rows,),
        in_specs=[part_spec, rows_spec, wspec, wspec, wspec, bspec],
        out_specs=rows_spec,
        out_shape=jax.ShapeDtypeStruct((N, D), jnp.float32),
    )(ah_p, zp, Wg2, W3[0:D], wd_pad, bd_pad)

    return out[:, :OUT]


# speedup vs baseline: 1.0565x; 1.0565x over previous
import functools

import jax
import jax.numpy as jnp
from jax import lax
from jax.experimental import pallas as pl
from jax.experimental.pallas import tpu as pltpu
from jax.experimental.pallas import tpu_sc as plsc

N = 10000
E = 320000
D = 128
OUT = 7

NC = 2
NS = 16
NW = NC * NS
K = 128
NCH0 = 110
NCH1 = 48
E0 = NS * NCH0 * K
E1 = NS * NCH1 * K
E_PAD = E0 + E1
RB = 80
NBLK = N // RB

_MESH = plsc.VectorSubcoreMesh(core_axis_name="c", subcore_axis_name="s")


def _scale_rows(rows_v, w_v):
    def scale(g, carry):
        wv = w_v[pl.ds(g * 16, 16)]
        base = g * 16
        for j in range(16):
            wk = wv[j]
            for d in range(D // 16):
                sl = pl.ds(d * 16, 16)
                rows_v[base + j, sl] = rows_v[base + j, sl] * wk
        return carry
    lax.fori_loop(0, K // 16, scale, 0)


def _run_core(pack_hbm, w_hbm, s_hbm, acc_sh, nch, sid,
              ip0, ip1, w0, w1, rows0, rows1, si0, si1, sg0, sg1):
    def _fire_idx(c, ip, wv, si):
        pltpu.async_copy(pack_hbm.at[sid, c], ip, si)
        pltpu.async_copy(w_hbm.at[sid, c], wv, si)

    def _wait_idx(ip, wv, si):
        pltpu.make_async_copy(pack_hbm.at[sid, 0], ip, si).wait()
        pltpu.make_async_copy(w_hbm.at[sid, 0], wv, si).wait()

    _fire_idx(0, ip0, w0, si0)
    _fire_idx(1, ip1, w1, si1)
    _wait_idx(ip0, w0, si0)
    pltpu.async_copy(s_hbm.at[ip0.at[1]], rows0, sg0)

    def _slot(c, ipA, wA, rowsA, sgA, siA, ipB, wB, rowsB, sgB, siB):
        @pl.when(c + 1 < nch)
        def _fire_next_gather():
            _wait_idx(ipB, wB, siB)
            pltpu.async_copy(s_hbm.at[ipB.at[1]], rowsB, sgB)

        pltpu.make_async_copy(s_hbm.at[ipA.at[1]], rowsA, sgA).wait()
        _scale_rows(rowsA, wA)
        pltpu.sync_copy(rowsA, acc_sh.at[ipA.at[0]], add=True)

        @pl.when(c + 2 < nch)
        def _fire_next_idx():
            _fire_idx(c + 2, ipA, wA, siA)

    def pipe(i, carry):
        c = 2 * i
        _slot(c, ip0, w0, rows0, sg0, si0, ip1, w1, rows1, sg1, si1)
        _slot(c + 1, ip1, w1, rows1, sg1, si1, ip0, w0, rows0, sg0, si0)
        return carry
    lax.fori_loop(0, nch // 2, pipe, 0)


def _sc_agg_body(s_hbm, pack0_hbm, w0_hbm, pack1_hbm, w1_hbm,
                 p_hbm, acc_sh, ip0, ip1, w0, w1, rows0, rows1,
                 si0, si1, sg0, sg1):
    cid = lax.axis_index("c")
    sid = lax.axis_index("s")

    z = jnp.zeros((16,), jnp.float32)

    def _zrow(i, carry):
        for d in range(D // 16):
            rows0[i, pl.ds(d * 16, 16)] = z
        return carry
    lax.fori_loop(0, RB, _zrow, 0)
    for i in range((NBLK + NS - 1) // NS):
        blk = sid + i * NS

        @pl.when(blk < NBLK)
        def _zero():
            pltpu.sync_copy(rows0.at[pl.ds(0, RB)], acc_sh.at[pl.ds(blk * RB, RB)])
    plsc.subcore_barrier()

    @pl.when(cid == 0)
    def _core0():
        _run_core(pack0_hbm, w0_hbm, s_hbm, acc_sh, NCH0, sid,
                  ip0, ip1, w0, w1, rows0, rows1, si0, si1, sg0, sg1)

    @pl.when(cid == 1)
    def _core1():
        _run_core(pack1_hbm, w1_hbm, s_hbm, acc_sh, NCH1, sid,
                  ip0, ip1, w0, w1, rows0, rows1, si0, si1, sg0, sg1)
    plsc.subcore_barrier()

    for i in range((NBLK + NS - 1) // NS):
        blk = sid + i * NS

        @pl.when(blk < NBLK)
        def _writeback():
            r0 = blk * RB
            pltpu.sync_copy(acc_sh.at[pl.ds(r0, RB)], p_hbm.at[cid, pl.ds(r0, RB)])


_sc_agg = functools.partial(
    pl.kernel,
    out_type=jax.ShapeDtypeStruct((NC, N, D), jnp.float32),
    mesh=_MESH,
    scratch_types=[
        pltpu.VMEM_SHARED((N, D), jnp.float32),
        pltpu.VMEM((2, K), jnp.int32),
        pltpu.VMEM((2, K), jnp.int32),
        pltpu.VMEM((K,), jnp.float32),
        pltpu.VMEM((K,), jnp.float32),
        pltpu.VMEM((K, D), jnp.float32),
        pltpu.VMEM((K, D), jnp.float32),
        pltpu.SemaphoreType.DMA,
        pltpu.SemaphoreType.DMA,
        pltpu.SemaphoreType.DMA,
        pltpu.SemaphoreType.DMA,
    ],
)(_sc_agg_body)


def _tc_mid_body(p_ref, x_ref, wg1_ref, wg3_ref, w1_ref, b1_ref,
                 h1_ref, hg2_ref, hl_ref):
    axc = p_ref[0] + p_ref[1]
    h1_ref[...] = jnp.maximum(
        jnp.dot(axc, wg1_ref[...], preferred_element_type=jnp.float32), 0.0)
    hg2_ref[...] = jnp.maximum(
        jnp.dot(axc, wg3_ref[...], preferred_element_type=jnp.float32), 0.0)
    hl_ref[...] = jnp.maximum(
        jnp.dot(x_ref[...], w1_ref[...], preferred_element_type=jnp.float32)
        + b1_ref[...], 0.0)


def _tc_pre_body(hg2_ref, hl_ref, w3_ref, b3_ref, zp_ref):
    zp_ref[...] = (
        jnp.dot(hg2_ref[...], w3_ref[0:D, :], preferred_element_type=jnp.float32)
        + jnp.dot(hl_ref[...], w3_ref[D:2 * D, :],
                  preferred_element_type=jnp.float32)
        + b3_ref[...])


def _tc_final_body(p_ref, zp_ref, wg2_ref, w3a_ref, wdp_ref, bdp_ref, out_ref):
    ahc = p_ref[0] + p_ref[1]
    hg1 = jnp.maximum(
        jnp.dot(ahc, wg2_ref[...], preferred_element_type=jnp.float32), 0.0)
    z = (jnp.dot(hg1, w3a_ref[...], preferred_element_type=jnp.float32)
         + zp_ref[...])
    c = jnp.dot(z, wdp_ref[...], preferred_element_type=jnp.float32) + bdp_ref[...]
    m = jnp.max(c, axis=1, keepdims=True)
    lse = jnp.log(jnp.sum(jnp.exp(c - m), axis=1, keepdims=True))
    out_ref[...] = c - m - lse


_TC_ROWS = 1000


def kernel(x, edge_index, edge_weight, Wg1, Wg2, Wg3, W1, b1, W3, b3, Wd, bd):
    row = edge_index[0].astype(jnp.int32)
    col = edge_index[1].astype(jnp.int32)
    w = edge_weight.astype(jnp.float32)

    pad = E_PAD - E
    zpad = jnp.zeros((pad,), jnp.int32)
    row_p = jnp.concatenate([row, zpad])
    col_p = jnp.concatenate([col, zpad])
    w_p = jnp.concatenate([w, jnp.zeros((pad,), jnp.float32)])
    pack0 = jnp.stack([row_p[:E0].reshape(NS, NCH0, K),
                       col_p[:E0].reshape(NS, NCH0, K)], axis=2)
    pack1 = jnp.stack([row_p[E0:].reshape(NS, NCH1, K),
                       col_p[E0:].reshape(NS, NCH1, K)], axis=2)
    w0p = w_p[:E0].reshape(NS, NCH0, K)
    w1p = w_p[E0:].reshape(NS, NCH1, K)

    ax_p = _sc_agg(x, pack0, w0p, pack1, w1p)

    grid_rows = N // _TC_ROWS
    full = lambda i: (0, 0)
    rows_spec = pl.BlockSpec((_TC_ROWS, D), lambda i: (i, 0))
    part_spec = pl.BlockSpec((NC, _TC_ROWS, D), lambda i: (0, i, 0))
    wspec = pl.BlockSpec((D, D), full)
    bspec = pl.BlockSpec((1, D), full)

    h1, hg2, hl = pl.pallas_call(
        _tc_mid_body,
        grid=(grid_rows,),
        in_specs=[part_spec, rows_spec, wspec, wspec, wspec, bspec],
        out_specs=[rows_spec, rows_spec, rows_spec],
        out_shape=[jax.ShapeDtypeStruct((N, D), jnp.float32)] * 3,
    )(ax_p, x, Wg1, Wg3, W1, b1.reshape(1, D))

    ah_p = _sc_agg(h1, pack0, w0p, pack1, w1p)

    zp = pl.pallas_call(
        _tc_pre_body,
        grid=(grid_rows,),
        in_specs=[rows_spec, rows_spec, pl.BlockSpec((2 * D, D), full), bspec],
        out_specs=rows_spec,
        out_shape=jax.ShapeDtypeStruct((N, D), jnp.float32),
    )(hg2, hl, W3[D:3 * D], b3.reshape(1, D))

    wd_pad = jnp.zeros((D, D), jnp.float32).at[:, :OUT].set(Wd)
    bd_pad = jnp.full((1, D), -1e30, jnp.float32).at[0, :OUT].set(bd)

    out = pl.pallas_call(
        _tc_final_body,
        grid=(grid_rows,),
        in_specs=[part_spec, rows_spec, wspec, wspec, wspec, bspec],
        out_specs=rows_spec,
        out_shape=jax.ShapeDtypeStruct((N, D), jnp.float32),
    )(ah_p, zp, Wg2, W3[0:D], wd_pad, bd_pad)

    return out[:, :OUT]
